# baseline (speedup 1.0000x reference)
import jax
import jax.numpy as jnp
from jax import lax
from jax.experimental import pallas as pl
from jax.experimental.pallas import tpu as pltpu
from jax.experimental.pallas import tpu_sc as plsc

VOCAB = 1000000
DIM = 64
MAXLEN = 200
BATCH = 4096

NC, NS, L = 2, 16, 16
NW = NC * NS
CB = 256
NBC = BATCH // CB
WPB = NW // NBC
RPW = MAXLEN // WPB


def _body(xT_hbm, tok2_hbm, pos_hbm, out_hbm,
          idx0, idx1, pidx0, pidx1,
          gb0, gb1, ob, pos_v,
          sg0, sg1, ss0, ss1):
    wid = lax.axis_index("s") * NC + lax.axis_index("c")
    bc = wid // WPB
    r0 = (wid % WPB) * RPW
    bbase = bc * CB
    idxs, pidxs = (idx0, idx1), (pidx0, pidx1)
    gbs = (gb0, gb1)
    iota = lax.iota(jnp.int32, L)
    sgs, sss = (sg0, sg1), (ss0, ss1)

    pltpu.sync_copy(pos_hbm, pos_v)

    def gather_start(t, slot):
        r = r0 + t
        pltpu.sync_copy(xT_hbm.at[r, pl.ds(bbase, CB)], idxs[slot])
        for j in range(CB // L):
            s = pl.ds(j * L, L)
            pidxs[slot][s] = lax.shift_right_logical(idxs[slot][s], 1)
        pltpu.async_copy(tok2_hbm.at[pidxs[slot]], gbs[slot], sgs[slot])

    def gather_wait(slot):
        pltpu.make_async_copy(tok2_hbm.at[pidxs[slot]], gbs[slot],
                              sgs[slot]).wait()

    def store_start(t):
        r = r0 + t
        pltpu.async_copy(ob, out_hbm.at[pl.ds(bbase, CB), r], ss0)

    def store_wait():
        pltpu.make_async_copy(ob, out_hbm.at[pl.ds(bbase, CB), 0],
                              ss0).wait()

    gather_start(0, 0)

    @pl.loop(0, RPW, step=2)
    def _pair(k):
        for b in range(2):
            cur = k + b
            nb = 1 - b

            @pl.when(cur + 1 < RPW)
            def _():
                gather_start(cur + 1, nb)

            gather_wait(b)

            @pl.when(cur >= 1)
            def _():
                store_wait()

            r = r0 + cur
            pv = [pos_v[pl.ds(r * DIM + c * L, L)] for c in range(DIM // L)]
            gb, idxv = gbs[b], idxs[b]

            @plsc.parallel_loop(0, CB, unroll=4)
            def _row(i):
                jb = (i // L) * L
                hv = lax.bitwise_and(idxv[pl.ds(jb, L)], 1)
                hs = jax.lax.gather(
                    hv, (iota * 0 + (i - jb))[:, None],
                    jax.lax.GatherDimensionNumbers(
                        offset_dims=(), collapsed_slice_dims=(0,),
                        start_index_map=(0,)),
                    (1,), mode=jax.lax.GatherScatterMode.PROMISE_IN_BOUNDS)
                m = hs == 1
                for c in range(DIM // L):
                    lo = gb[i, pl.ds(c * L, L)]
                    hi = gb[i, pl.ds(DIM + c * L, L)]
                    ob[i, pl.ds(c * L, L)] = jnp.where(m, hi, lo) + pv[c]

            store_start(cur)

    store_wait()


@jax.jit
def _run(xT, tok2, posf):
    mesh = plsc.VectorSubcoreMesh(core_axis_name="c", subcore_axis_name="s")
    return pl.kernel(
        _body,
        out_type=jax.ShapeDtypeStruct((BATCH, MAXLEN, DIM), jnp.float32),
        mesh=mesh,
        scratch_types=[
            pltpu.VMEM((CB,), jnp.int32),
            pltpu.VMEM((CB,), jnp.int32),
            pltpu.VMEM((CB,), jnp.int32),
            pltpu.VMEM((CB,), jnp.int32),
            pltpu.VMEM((CB, 2 * DIM), jnp.float32),
            pltpu.VMEM((CB, 2 * DIM), jnp.float32),
            pltpu.VMEM((CB, DIM), jnp.float32),
            pltpu.VMEM((MAXLEN * DIM,), jnp.float32),
            pltpu.SemaphoreType.DMA,
            pltpu.SemaphoreType.DMA,
            pltpu.SemaphoreType.DMA,
            pltpu.SemaphoreType.DMA,
        ],
        compiler_params=pltpu.CompilerParams(use_tc_tiling_on_sc=True,
                                             needs_layout_passes=False),
    )(xT, tok2, posf)


def kernel(x, token_table, pos_table):
    xT = jnp.swapaxes(x.astype(jnp.int32), 0, 1)
    half = VOCAB // 2
    tok2 = jnp.concatenate(
        [token_table[:half].reshape(half // 2, 2 * DIM),
         token_table[half:].reshape(half // 2, 2 * DIM)], axis=0)
    return _run(xT, tok2, pos_table.reshape(-1))


# speedup vs baseline: 1.3760x; 1.3760x over previous
import jax
import jax.numpy as jnp
from jax import lax
from jax.experimental import pallas as pl
from jax.experimental.pallas import tpu as pltpu
from jax.experimental.pallas import tpu_sc as plsc

VOCAB = 1000000
DIM = 64
MAXLEN = 200
BATCH = 4096

NC, NS, L = 2, 16, 16
NW = NC * NS
CB = 256
NBC = BATCH // CB
WPB = NW // NBC
RPW = MAXLEN // WPB


def _body(xT_hbm, tok2_hbm, pos_hbm, out_hbm,
          idx0, idx1, pidx0, pidx1,
          gb0, gb1, ob, pos_v,
          sg0, sg1, ss0, ss1):
    wid = lax.axis_index("s") * NC + lax.axis_index("c")
    bc = wid // WPB
    r0 = (wid % WPB) * RPW
    bbase = bc * CB
    idxs, pidxs = (idx0, idx1), (pidx0, pidx1)
    gbs = (gb0, gb1)
    iota = lax.iota(jnp.int32, L)
    sgs, sss = (sg0, sg1), (ss0, ss1)

    pltpu.sync_copy(pos_hbm, pos_v)

    def gather_start(t, slot):
        r = r0 + t
        pltpu.sync_copy(xT_hbm.at[r, pl.ds(bbase, CB)], idxs[slot])
        for j in range(CB // L):
            s = pl.ds(j * L, L)
            pidxs[slot][s] = lax.shift_right_logical(idxs[slot][s], 1)
        pltpu.async_copy(tok2_hbm.at[pidxs[slot]], gbs[slot], sgs[slot])

    def gather_wait(slot):
        pltpu.make_async_copy(tok2_hbm.at[pidxs[slot]], gbs[slot],
                              sgs[slot]).wait()

    def store_start(t):
        r = r0 + t
        pltpu.async_copy(ob, out_hbm.at[pl.ds(bbase, CB), r], ss0)

    def store_wait():
        pltpu.make_async_copy(ob, out_hbm.at[pl.ds(bbase, CB), 0],
                              ss0).wait()

    gather_start(0, 0)

    @pl.loop(0, RPW, step=2)
    def _pair(k):
        for b in range(2):
            cur = k + b
            nb = 1 - b

            @pl.when(cur + 1 < RPW)
            def _():
                gather_start(cur + 1, nb)

            gather_wait(b)

            @pl.when(cur >= 1)
            def _():
                store_wait()

            r = r0 + cur
            pv = [pos_v[pl.ds(r * DIM + c * L, L)] for c in range(DIM // L)]
            gb, idxv = gbs[b], idxs[b]

            @plsc.parallel_loop(0, CB, unroll=4)
            def _row(i):
                jb = (i // L) * L
                hv = lax.bitwise_and(idxv[pl.ds(jb, L)], 1)
                hs = jax.lax.gather(
                    hv, (iota * 0 + (i - jb))[:, None],
                    jax.lax.GatherDimensionNumbers(
                        offset_dims=(), collapsed_slice_dims=(0,),
                        start_index_map=(0,)),
                    (1,), mode=jax.lax.GatherScatterMode.PROMISE_IN_BOUNDS)
                m = hs == 1
                for c in range(DIM // L):
                    lo = gb[i, pl.ds(c * L, L)]
                    hi = gb[i, pl.ds(DIM + c * L, L)]
                    ob[i, pl.ds(c * L, L)] = jnp.where(m, hi, lo) + pv[c]

            store_start(cur)

    store_wait()


@jax.jit
def _run(xT, tok2, posf):
    mesh = plsc.VectorSubcoreMesh(core_axis_name="c", subcore_axis_name="s")
    return pl.kernel(
        _body,
        out_type=jax.ShapeDtypeStruct((BATCH, MAXLEN, DIM), jnp.float32),
        mesh=mesh,
        scratch_types=[
            pltpu.VMEM((CB,), jnp.int32),
            pltpu.VMEM((CB,), jnp.int32),
            pltpu.VMEM((CB,), jnp.int32),
            pltpu.VMEM((CB,), jnp.int32),
            pltpu.VMEM((CB, 2 * DIM), jnp.float32),
            pltpu.VMEM((CB, 2 * DIM), jnp.float32),
            pltpu.VMEM((CB, DIM), jnp.float32),
            pltpu.VMEM((MAXLEN * DIM,), jnp.float32),
            pltpu.SemaphoreType.DMA,
            pltpu.SemaphoreType.DMA,
            pltpu.SemaphoreType.DMA,
            pltpu.SemaphoreType.DMA,
        ],
        compiler_params=pltpu.CompilerParams(use_tc_tiling_on_sc=True,
                                             needs_layout_passes=False),
    )(xT, tok2, posf)


def kernel(x, token_table, pos_table):
    xT = jnp.swapaxes(x.astype(jnp.int32), 0, 1)
    tok2 = token_table.reshape(VOCAB // 2, 2 * DIM)
    return _run(xT, tok2, pos_table.reshape(-1))
